# baseline (speedup 1.0000x reference)
import functools

import jax
import jax.numpy as jnp
from jax import lax
from jax.experimental import pallas as pl
from jax.experimental.pallas import tpu as pltpu
from jax.experimental.pallas import tpu_sc as plsc

_NC = 2
_NS = 16
_NW = _NC * _NS
_LANES = 16
_CH = 2


def _add_pos(ibuf, obuf, posv, n):
    def body(j, _):
        off = j * _LANES
        pv = posv[pl.ds(off, _LANES)]
        for r in range(_CH):
            obuf[pl.ds(r * n + off, _LANES)] = ibuf[pl.ds(r * n + off, _LANES)] + pv
        return 0

    lax.fori_loop(0, n // _LANES, body, 0)


def _sc_body(x_hbm, pos_hbm, out_hbm, posv, ibufs, obufs, isems, osems, n, nch):
    wid = lax.axis_index("s") * _NC + lax.axis_index("c")
    base = wid * nch * _CH * n
    chunk = _CH * n

    def in_slice(c):
        return x_hbm.at[pl.ds(base + c * chunk, chunk)]

    def out_slice(c):
        return out_hbm.at[pl.ds(base + c * chunk, chunk)]

    pltpu.sync_copy(pos_hbm, posv)

    pltpu.async_copy(in_slice(0), ibufs[0], isems[0])
    pltpu.async_copy(in_slice(1), ibufs[1], isems[1])

    def step(c, cur, first):
        pltpu.make_async_copy(in_slice(c), ibufs[cur], isems[cur]).wait()
        if not first:
            pltpu.make_async_copy(obufs[cur], out_slice(c - 2), osems[cur]).wait()
        _add_pos(ibufs[cur], obufs[cur], posv, n)
        pltpu.async_copy(obufs[cur], out_slice(c), osems[cur])

    for cur in range(2):
        step(cur, cur, True)
        pltpu.async_copy(in_slice(cur + 2), ibufs[cur], isems[cur])

    def outer(k, _):
        for cur in range(2):
            c = 2 * k + cur
            step(c, cur, False)
            pltpu.async_copy(in_slice(c + 2), ibufs[cur], isems[cur])
        return 0

    lax.fori_loop(1, nch // 2 - 1, outer, 0)

    for cur in range(2):
        step(nch - 2 + cur, cur, False)

    for cur in range(2):
        pltpu.make_async_copy(obufs[cur], out_slice(nch - 2 + cur), osems[cur]).wait()


def kernel(x, pos_table):
    B, L, D = x.shape
    N = L * D
    nch = B // _NW // _CH
    mesh = plsc.VectorSubcoreMesh(core_axis_name="c", subcore_axis_name="s")
    body = functools.partial(_sc_body, n=N, nch=nch)
    run = pl.kernel(
        body,
        out_type=jax.ShapeDtypeStruct((B * N,), x.dtype),
        mesh=mesh,
        scratch_types=[
            pltpu.VMEM((N,), x.dtype),
            [pltpu.VMEM((_CH * N,), x.dtype) for _ in range(2)],
            [pltpu.VMEM((_CH * N,), x.dtype) for _ in range(2)],
            [pltpu.SemaphoreType.DMA for _ in range(2)],
            [pltpu.SemaphoreType.DMA for _ in range(2)],
        ],
    )
    out = run(x.reshape(B * N), pos_table.reshape(N))
    return out.reshape(B, L, D)


# speedup vs baseline: 1.3137x; 1.3137x over previous
import functools

import jax
import jax.numpy as jnp
from jax import lax
from jax.experimental import pallas as pl
from jax.experimental.pallas import tpu as pltpu
from jax.experimental.pallas import tpu_sc as plsc

_NC = 2
_NS = 16
_NW = _NC * _NS
_LANES = 16
_CH = 2
_NBUF = 4


def _add_pos(buf, posv, n):
    @plsc.parallel_loop(0, n // _LANES, unroll=8)
    def _(j):
        off = j * _LANES
        pv = posv[pl.ds(off, _LANES)]
        for r in range(_CH):
            plsc.addupdate(buf.at[pl.ds(r * n + off, _LANES)], pv)


def _sc_body(x_hbm, pos_hbm, out_hbm, posv, bufs, isems, osems, n, nch):
    wid = lax.axis_index("s") * _NC + lax.axis_index("c")
    base = wid * nch * _CH * n
    chunk = _CH * n

    def in_slice(c):
        return x_hbm.at[pl.ds(base + c * chunk, chunk)]

    def out_slice(c):
        return out_hbm.at[pl.ds(base + c * chunk, chunk)]

    pltpu.sync_copy(pos_hbm, posv)

    def start_in(c, cur):
        pltpu.async_copy(in_slice(c), bufs[cur], isems[cur])

    def wait_in(c, cur):
        pltpu.make_async_copy(in_slice(c), bufs[cur], isems[cur]).wait()

    def start_out(c, cur):
        pltpu.async_copy(bufs[cur], out_slice(c), osems[cur])

    def wait_out(c, cur):
        pltpu.make_async_copy(bufs[cur], out_slice(c), osems[cur]).wait()

    def process(c, cur, wait_prev, prefetch):
        wait_in(c, cur)
        _add_pos(bufs[cur], posv, n)
        start_out(c, cur)
        if wait_prev:
            wait_out(c - 2, (cur + 2) % _NBUF)
        if prefetch:
            start_in(c + 2, (cur + 2) % _NBUF)

    start_in(0, 0)
    start_in(1, 1)
    for cur in range(_NBUF):
        process(cur, cur, wait_prev=cur >= 2, prefetch=True)

    def outer(k, _):
        c0 = _NBUF * k
        for cur in range(_NBUF):
            process(c0 + cur, cur, wait_prev=True, prefetch=True)
        return 0

    lax.fori_loop(1, nch // _NBUF - 1, outer, 0)

    c0 = nch - _NBUF
    for cur in range(_NBUF):
        process(c0 + cur, cur, wait_prev=True, prefetch=cur < 2)

    for cur in range(2):
        wait_out(nch - 2 + cur, (cur + 2) % _NBUF)


def kernel(x, pos_table):
    B, L, D = x.shape
    N = L * D
    nch = B // _NW // _CH
    mesh = plsc.VectorSubcoreMesh(core_axis_name="c", subcore_axis_name="s")
    body = functools.partial(_sc_body, n=N, nch=nch)
    run = pl.kernel(
        body,
        out_type=jax.ShapeDtypeStruct((B * N,), x.dtype),
        mesh=mesh,
        scratch_types=[
            pltpu.VMEM((N,), x.dtype),
            [pltpu.VMEM((_CH * N,), x.dtype) for _ in range(_NBUF)],
            [pltpu.SemaphoreType.DMA for _ in range(_NBUF)],
            [pltpu.SemaphoreType.DMA for _ in range(_NBUF)],
        ],
    )
    out = run(x.reshape(B * N), pos_table.reshape(N))
    return out.reshape(B, L, D)


# speedup vs baseline: 2.7550x; 2.0972x over previous
import functools

import jax
import jax.numpy as jnp
from jax import lax
from jax.experimental import pallas as pl
from jax.experimental.pallas import tpu as pltpu
from jax.experimental.pallas import tpu_sc as plsc

_NC = 2
_NS = 16
_NW = _NC * _NS
_LANES = 16
_ROWS = 8
_COLS = 3200
_NBUF = 4


def _add_pos(buf, posv, col0):
    @plsc.parallel_loop(0, (_COLS // 128) * 8, unroll=2)
    def _(i):
        c = (i // 8) * 128 + (i % 8) * _LANES
        pv = posv[pl.ds(col0 + c, _LANES)]
        for r in range(_ROWS):
            plsc.addupdate(buf.at[r, pl.ds(c, _LANES)], pv)


def _sc_body(x_hbm, pos_hbm, out_hbm, posv, bufs, isems, osems, n, nch):
    wid = lax.axis_index("s") * _NC + lax.axis_index("c")
    panels = n // _COLS

    def chunk_slices(c):
        g = c // panels
        p = c % panels
        rows = pl.ds((wid * (nch // panels) + g) * _ROWS, _ROWS)
        cols = pl.ds(p * _COLS, _COLS)
        return rows, cols, p * _COLS

    pltpu.sync_copy(pos_hbm, posv)

    def start_in(c, cur):
        rows, cols, _ = chunk_slices(c)
        pltpu.async_copy(x_hbm.at[rows, cols], bufs[cur], isems[cur])

    def wait_in(c, cur):
        rows, cols, _ = chunk_slices(c)
        pltpu.make_async_copy(x_hbm.at[rows, cols], bufs[cur], isems[cur]).wait()

    def start_out(c, cur):
        rows, cols, _ = chunk_slices(c)
        pltpu.async_copy(bufs[cur], out_hbm.at[rows, cols], osems[cur])

    def wait_out(c, cur):
        rows, cols, _ = chunk_slices(c)
        pltpu.make_async_copy(bufs[cur], out_hbm.at[rows, cols], osems[cur]).wait()

    def process(c, cur, wait_prev, prefetch):
        wait_in(c, cur)
        _, _, col0 = chunk_slices(c)
        _add_pos(bufs[cur], posv, col0)
        start_out(c, cur)
        if wait_prev:
            wait_out(c - 2, (cur + 2) % _NBUF)
        if prefetch:
            start_in(c + 2, (cur + 2) % _NBUF)

    start_in(0, 0)
    start_in(1, 1)
    for cur in range(_NBUF):
        process(cur, cur, wait_prev=cur >= 2, prefetch=True)

    def outer(k, _):
        c0 = _NBUF * k
        for cur in range(_NBUF):
            process(c0 + cur, cur, wait_prev=True, prefetch=True)
        return 0

    lax.fori_loop(1, nch // _NBUF - 1, outer, 0)

    c0 = nch - _NBUF
    for cur in range(_NBUF):
        process(c0 + cur, cur, wait_prev=True, prefetch=cur < 2)

    for cur in range(2):
        wait_out(nch - 2 + cur, (cur + 2) % _NBUF)


def kernel(x, pos_table):
    B, L, D = x.shape
    N = L * D
    panels = N // _COLS
    nch = (B // _NW // _ROWS) * panels
    mesh = plsc.VectorSubcoreMesh(core_axis_name="c", subcore_axis_name="s")
    body = functools.partial(_sc_body, n=N, nch=nch)
    run = pl.kernel(
        body,
        out_type=jax.ShapeDtypeStruct((B, N), x.dtype),
        mesh=mesh,
        compiler_params=pltpu.CompilerParams(use_tc_tiling_on_sc=True),
        scratch_types=[
            pltpu.VMEM((N,), x.dtype),
            [pltpu.VMEM((_ROWS, _COLS), x.dtype) for _ in range(_NBUF)],
            [pltpu.SemaphoreType.DMA for _ in range(_NBUF)],
            [pltpu.SemaphoreType.DMA for _ in range(_NBUF)],
        ],
    )
    out = run(x.reshape(B, N), pos_table.reshape(N))
    return out.reshape(B, L, D)


# speedup vs baseline: 2.9622x; 1.0752x over previous
import jax
import jax.numpy as jnp
from jax.experimental import pallas as pl
from jax.experimental.pallas import tpu as pltpu

_NCHUNK = 64
_NBUF = 6


def _body(x_hbm, pos_vmem, o_hbm, ibuf, obuf, isem, osem):
    i = pl.program_id(0)
    nrows = x_hbm.shape[0] // _NCHUNK
    slot = jax.lax.rem(i, _NBUF)

    def in_copy(step, sl):
        return pltpu.make_async_copy(
            x_hbm.at[pl.ds(step * nrows, nrows), :], ibuf.at[sl], isem.at[sl]
        )

    def out_copy(step, sl):
        return pltpu.make_async_copy(
            obuf.at[sl], o_hbm.at[pl.ds(step * nrows, nrows), :], osem.at[sl]
        )

    def start_in(step, sl, prio):
        pltpu.async_copy(
            x_hbm.at[pl.ds(step * nrows, nrows), :], ibuf.at[sl], isem.at[sl],
            priority=prio,
        )

    def start_out(step, sl, prio):
        pltpu.async_copy(
            obuf.at[sl], o_hbm.at[pl.ds(step * nrows, nrows), :], osem.at[sl],
            priority=prio,
        )

    @pl.when(i == 0)
    def _prologue():
        for k in range(_NBUF):
            start_in(k, k, 1)

    in_copy(i, slot).wait()

    @pl.when(i >= _NBUF)
    def _reclaim():
        out_copy(i - _NBUF, slot).wait()

    obuf[slot] = ibuf[slot] + pos_vmem[...]

    start_out(i, slot, 0)

    @pl.when(i + _NBUF < _NCHUNK)
    def _prefetch():
        start_in(i + _NBUF, slot, 1)

    @pl.when(i == _NCHUNK - 1)
    def _drain():
        for k in range(_NBUF):
            out_copy(_NCHUNK - _NBUF + k, k).wait()


def kernel(x, pos_table):
    B, L, D = x.shape
    N = L * D
    nrows = B // _NCHUNK
    out = pl.pallas_call(
        _body,
        grid=(_NCHUNK,),
        in_specs=[
            pl.BlockSpec(memory_space=pl.ANY),
            pl.BlockSpec((1, N), lambda i: (0, 0)),
        ],
        out_specs=pl.BlockSpec(memory_space=pl.ANY),
        out_shape=jax.ShapeDtypeStruct((B, N), x.dtype),
        scratch_shapes=[
            pltpu.VMEM((_NBUF, nrows, N), x.dtype),
            pltpu.VMEM((_NBUF, nrows, N), x.dtype),
            pltpu.SemaphoreType.DMA((_NBUF,)),
            pltpu.SemaphoreType.DMA((_NBUF,)),
        ],
    )(x.reshape(B, N), pos_table.reshape(1, N))
    return out.reshape(B, L, D)
